# baseline (speedup 1.0000x reference)
import jax
import jax.numpy as jnp
from jax.experimental import pallas as pl
from jax.experimental.pallas import tpu as pltpu

_F = 1024
_ROWS = 512


def _mask_fill_body(idx_ref, z_ref, zo_ref, m_ref):
    idx = idx_ref[0, 0, :]
    col = jax.lax.broadcasted_iota(jnp.int32, (_ROWS, _F), 1)
    mask = col > idx[:, None]
    m_ref[...] = mask
    zo_ref[...] = jnp.where(mask, jnp.zeros_like(z_ref[...]), z_ref[...])


def kernel(z):
    b, s, f = z.shape
    tokens = b * s
    idx = jax.random.randint(jax.random.key(1), (b, s), 0, f)
    g = tokens // _ROWS
    idx3 = idx.reshape(g, 1, _ROWS).astype(jnp.int32)
    z2 = z.reshape(tokens, f)
    zm, mask = pl.pallas_call(
        _mask_fill_body,
        grid=(g,),
        in_specs=[
            pl.BlockSpec((1, 1, _ROWS), lambda i: (i, 0, 0)),
            pl.BlockSpec((_ROWS, f), lambda i: (i, 0)),
        ],
        out_specs=[
            pl.BlockSpec((_ROWS, f), lambda i: (i, 0)),
            pl.BlockSpec((_ROWS, f), lambda i: (i, 0)),
        ],
        out_shape=[
            jax.ShapeDtypeStruct((tokens, f), z.dtype),
            jax.ShapeDtypeStruct((tokens, f), jnp.bool_),
        ],
        compiler_params=pltpu.CompilerParams(
            dimension_semantics=("parallel",),
        ),
    )(idx3, z2)
    return zm.reshape(b, s, f), mask.reshape(b, s, f)


# speedup vs baseline: 1.0643x; 1.0643x over previous
import jax
import jax.numpy as jnp
from jax.experimental import pallas as pl
from jax.experimental.pallas import tpu as pltpu

_F = 1024
_ROWS = 2048


def _mask_fill_body(idx_ref, z_ref, zo_ref, m_ref):
    idx = idx_ref[0, 0, :]
    col = jax.lax.broadcasted_iota(jnp.int32, (_ROWS, _F), 1)
    mask = col > idx[:, None]
    m_ref[...] = mask
    zo_ref[...] = jnp.where(mask, jnp.zeros_like(z_ref[...]), z_ref[...])


def kernel(z):
    b, s, f = z.shape
    tokens = b * s
    idx = jax.random.randint(jax.random.key(1), (b, s), 0, f)
    g = tokens // _ROWS
    idx3 = idx.reshape(g, 1, _ROWS).astype(jnp.int32)
    z2 = z.reshape(tokens, f)
    zm, mask = pl.pallas_call(
        _mask_fill_body,
        grid=(g,),
        in_specs=[
            pl.BlockSpec((1, 1, _ROWS), lambda i: (i, 0, 0)),
            pl.BlockSpec((_ROWS, f), lambda i: (i, 0)),
        ],
        out_specs=[
            pl.BlockSpec((_ROWS, f), lambda i: (i, 0)),
            pl.BlockSpec((_ROWS, f), lambda i: (i, 0)),
        ],
        out_shape=[
            jax.ShapeDtypeStruct((tokens, f), z.dtype),
            jax.ShapeDtypeStruct((tokens, f), jnp.bool_),
        ],
        compiler_params=pltpu.CompilerParams(
            dimension_semantics=("parallel",),
        ),
    )(idx3, z2)
    return zm.reshape(b, s, f), mask.reshape(b, s, f)


# speedup vs baseline: 1.4701x; 1.3813x over previous
import jax
import jax.numpy as jnp
from jax.experimental import pallas as pl
from jax.experimental.pallas import tpu as pltpu

_F = 1024
_ROWS = 2048


def _mask_fill_body(idx_ref, z_ref, zo_ref, m_ref):
    idx = idx_ref[0, 0, :]
    col = jax.lax.broadcasted_iota(jnp.int32, (_ROWS, _F), 1)
    mask = col > idx[:, None]
    m_ref[...] = mask.astype(jnp.int8)
    zo_ref[...] = jnp.where(mask, jnp.zeros_like(z_ref[...]), z_ref[...])


def kernel(z):
    b, s, f = z.shape
    tokens = b * s
    idx = jax.random.randint(jax.random.key(1), (b, s), 0, f)
    g = tokens // _ROWS
    idx3 = idx.reshape(g, 1, _ROWS).astype(jnp.int32)
    z2 = z.reshape(tokens, f)
    zm, mask8 = pl.pallas_call(
        _mask_fill_body,
        grid=(g,),
        in_specs=[
            pl.BlockSpec((1, 1, _ROWS), lambda i: (i, 0, 0)),
            pl.BlockSpec((_ROWS, f), lambda i: (i, 0)),
        ],
        out_specs=[
            pl.BlockSpec((_ROWS, f), lambda i: (i, 0)),
            pl.BlockSpec((_ROWS, f), lambda i: (i, 0)),
        ],
        out_shape=[
            jax.ShapeDtypeStruct((tokens, f), z.dtype),
            jax.ShapeDtypeStruct((tokens, f), jnp.int8),
        ],
        compiler_params=pltpu.CompilerParams(
            dimension_semantics=("parallel",),
        ),
    )(idx3, z2)
    mask = mask8.astype(jnp.bool_)
    return zm.reshape(b, s, f), mask.reshape(b, s, f)
